# Initial kernel scaffold; baseline (speedup 1.0000x reference)
import jax
import jax.numpy as jnp
from jax.experimental import pallas as pl


def kernel(ui_graph, iu_graph, image_ui_graph, image_iu_graph, text_ui_graph, text_iu_graph, image_feats_raw, text_feats_raw, W_img, b_img, W_txt, b_txt, user_id_emb, item_id_emb, w_q, w_k, w_cat):
    raise NotImplementedError("write your pallas kernel here")


# speedup vs baseline: 1.2679x; 1.2679x over previous
import jax
import jax.numpy as jnp
from jax.experimental import pallas as pl
from jax.experimental.pallas import tpu as pltpu

N = 4096
EMBED = 64
HEAD_NUM = 4
D_H = EMBED // HEAD_NUM
MODEL_CAT_RATE = 0.02
ID_CAT_RATE = 0.36
IMG_DIM = 4096
TXT_DIM = 1024

_F32 = jnp.float32


def _dot(a, b):
    return jax.lax.dot_general(a, b, (((1,), (0,)), ((), ())),
                               preferred_element_type=_F32)


def _rownorm(x):
    n = jnp.sqrt(jnp.sum(x * x, axis=1, keepdims=True))
    return x / jnp.maximum(n, 1e-12)


def _mhsa_mean(a, b, w_q, w_k, w_cat):
    qa = _dot(a, w_q)
    qb = _dot(b, w_q)
    ka = _dot(a, w_k)
    kb = _dot(b, w_k)
    scale = 1.0 / jnp.sqrt(jnp.float32(D_H))
    z_parts_a = []
    z_parts_b = []
    for h in range(HEAD_NUM):
        s = slice(h * D_H, (h + 1) * D_H)
        qah, qbh = qa[:, s], qb[:, s]
        kah, kbh = ka[:, s], kb[:, s]
        l_aa = jnp.sum(qah * kah, axis=1, keepdims=True) * scale
        l_ab = jnp.sum(qah * kbh, axis=1, keepdims=True) * scale
        l_ba = jnp.sum(qbh * kah, axis=1, keepdims=True) * scale
        l_bb = jnp.sum(qbh * kbh, axis=1, keepdims=True) * scale
        m_a = jnp.maximum(l_aa, l_ab)
        e_aa = jnp.exp(l_aa - m_a)
        e_ab = jnp.exp(l_ab - m_a)
        za = (e_aa * a + e_ab * b) / (e_aa + e_ab)
        m_b = jnp.maximum(l_ba, l_bb)
        e_ba = jnp.exp(l_ba - m_b)
        e_bb = jnp.exp(l_bb - m_b)
        zb = (e_ba * a + e_bb * b) / (e_ba + e_bb)
        z_parts_a.append(za)
        z_parts_b.append(zb)
    zcat_a = jnp.concatenate(z_parts_a, axis=1)
    zcat_b = jnp.concatenate(z_parts_b, axis=1)
    out_a = _dot(zcat_a, w_cat)
    out_b = _dot(zcat_b, w_cat)
    return 0.5 * (out_a + out_b)


def _stage_a_kernel(img_ui_ref, txt_ui_ref, img_iu_ref, txt_iu_ref,
                    img_raw_ref, txt_raw_ref,
                    w_img_ref, b_img_ref, w_txt_ref, b_txt_ref,
                    uemb_full_ref, iemb_full_ref,
                    uemb_blk_ref, iemb_blk_ref,
                    w_q_ref, w_k_ref, w_cat_ref,
                    img_feats_ref, txt_feats_ref,
                    iu_id_ref, tu_id_ref, ug0_ref, ig0_ref):
    img_feats_ref[...] = _dot(img_raw_ref[...], w_img_ref[...]) + b_img_ref[...]
    txt_feats_ref[...] = _dot(txt_raw_ref[...], w_txt_ref[...]) + b_txt_ref[...]
    iemb_full = iemb_full_ref[...]
    uemb_full = uemb_full_ref[...]
    iu_id = _dot(img_ui_ref[...], iemb_full)
    tu_id = _dot(txt_ui_ref[...], iemb_full)
    ii_id = _dot(img_iu_ref[...], uemb_full)
    ti_id = _dot(txt_iu_ref[...], uemb_full)
    iu_id_ref[...] = iu_id
    tu_id_ref[...] = tu_id
    w_q, w_k, w_cat = w_q_ref[...], w_k_ref[...], w_cat_ref[...]
    user_emb = _mhsa_mean(iu_id, tu_id, w_q, w_k, w_cat)
    item_emb = _mhsa_mean(ii_id, ti_id, w_q, w_k, w_cat)
    ug0_ref[...] = uemb_blk_ref[...] + ID_CAT_RATE * _rownorm(user_emb)
    ig0_ref[...] = iemb_blk_ref[...] + ID_CAT_RATE * _rownorm(item_emb)


def _spmm_kernel(g_ref, rhs_ref, out_ref):
    out_ref[...] = _dot(g_ref[...], rhs_ref[...])


def _stage_e_kernel(g_ref, dfull_ref, ug0_ref, cpack_ref, ug2_ref, ufin_ref):
    t = _dot(g_ref[...], dfull_ref[:, 2 * EMBED:3 * EMBED])
    ug2 = jax.nn.softmax(t, axis=-1)
    ug2_ref[...] = ug2
    cpack = cpack_ref[...]
    iuf = cpack[:, 0:EMBED]
    tuf = cpack[:, EMBED:2 * EMBED]
    ug1 = cpack[:, 2 * EMBED:3 * EMBED]
    u = (ug0_ref[...] + ug1 + ug2) * (1.0 / 3.0)
    ufin_ref[...] = (u + MODEL_CAT_RATE * _rownorm(iuf)
                     + MODEL_CAT_RATE * _rownorm(tuf))


def _stage_f_kernel(g_ref, ug2_ref, ig0_ref, dpack_ref, ifin_ref):
    t = _dot(g_ref[...], ug2_ref[...])
    ig2 = jax.nn.softmax(t, axis=-1)
    dpack = dpack_ref[...]
    iif = dpack[:, 0:EMBED]
    tif = dpack[:, EMBED:2 * EMBED]
    ig1 = dpack[:, 2 * EMBED:3 * EMBED]
    i = (ig0_ref[...] + ig1 + ig2) * (1.0 / 3.0)
    ifin_ref[...] = (i + MODEL_CAT_RATE * _rownorm(iif)
                     + MODEL_CAT_RATE * _rownorm(tif))


def _row_spec(r, cols):
    return pl.BlockSpec((r, cols), lambda b: (b, 0))


def _full_spec(rows, cols):
    return pl.BlockSpec((rows, cols), lambda b: (0, 0))


_ARB = pltpu.CompilerParams(dimension_semantics=("arbitrary",))


def kernel(ui_graph, iu_graph, image_ui_graph, image_iu_graph, text_ui_graph,
           text_iu_graph, image_feats_raw, text_feats_raw, W_img, b_img,
           W_txt, b_txt, user_id_emb, item_id_emb, w_q, w_k, w_cat):
    f32 = _F32
    b_img2 = b_img.reshape(1, EMBED)
    b_txt2 = b_txt.reshape(1, EMBED)

    RA = 256
    (image_feats, text_feats, image_user_id, text_user_id,
     u_g0, i_g0) = pl.pallas_call(
        _stage_a_kernel,
        grid=(N // RA,),
        in_specs=[
            _row_spec(RA, N),
            _row_spec(RA, N),
            _row_spec(RA, N),
            _row_spec(RA, N),
            _row_spec(RA, IMG_DIM),
            _row_spec(RA, TXT_DIM),
            _full_spec(IMG_DIM, EMBED),
            _full_spec(1, EMBED),
            _full_spec(TXT_DIM, EMBED),
            _full_spec(1, EMBED),
            _full_spec(N, EMBED),
            _full_spec(N, EMBED),
            _row_spec(RA, EMBED),
            _row_spec(RA, EMBED),
            _full_spec(EMBED, EMBED),
            _full_spec(EMBED, EMBED),
            _full_spec(HEAD_NUM * EMBED, EMBED),
        ],
        out_specs=[_row_spec(RA, EMBED)] * 6,
        out_shape=[jax.ShapeDtypeStruct((N, EMBED), f32)] * 6,
        compiler_params=_ARB,
    )(image_ui_graph, text_ui_graph, image_iu_graph, text_iu_graph,
      image_feats_raw, text_feats_raw, W_img, b_img2, W_txt, b_txt2,
      user_id_emb, item_id_emb, user_id_emb, item_id_emb, w_q, w_k, w_cat)

    RC = 512
    rhs_c = jnp.concatenate([image_feats, text_feats, i_g0], axis=1)
    cpack = pl.pallas_call(
        _spmm_kernel,
        grid=(N // RC,),
        in_specs=[_row_spec(RC, N), _full_spec(N, 3 * EMBED)],
        out_specs=_row_spec(RC, 3 * EMBED),
        out_shape=jax.ShapeDtypeStruct((N, 3 * EMBED), f32),
        compiler_params=_ARB,
    )(ui_graph, rhs_c)

    dpack = pl.pallas_call(
        _spmm_kernel,
        grid=(N // RC,),
        in_specs=[_row_spec(RC, N), _full_spec(N, 3 * EMBED)],
        out_specs=_row_spec(RC, 3 * EMBED),
        out_shape=jax.ShapeDtypeStruct((N, 3 * EMBED), f32),
        compiler_params=_ARB,
    )(iu_graph, cpack)

    u_g2, u_g = pl.pallas_call(
        _stage_e_kernel,
        grid=(N // RC,),
        in_specs=[_row_spec(RC, N), _full_spec(N, 3 * EMBED), _row_spec(RC, EMBED),
                  _row_spec(RC, 3 * EMBED)],
        out_specs=[_row_spec(RC, EMBED)] * 2,
        out_shape=[jax.ShapeDtypeStruct((N, EMBED), f32)] * 2,
        compiler_params=_ARB,
    )(ui_graph, dpack, u_g0, cpack)

    i_g = pl.pallas_call(
        _stage_f_kernel,
        grid=(N // RC,),
        in_specs=[_row_spec(RC, N), _full_spec(N, EMBED),
                  _row_spec(RC, EMBED), _row_spec(RC, 3 * EMBED)],
        out_specs=_row_spec(RC, EMBED),
        out_shape=jax.ShapeDtypeStruct((N, EMBED), f32),
        compiler_params=_ARB,
    )(iu_graph, u_g2, i_g0, dpack)

    image_user_feats = cpack[:, 0:EMBED]
    text_user_feats = cpack[:, EMBED:2 * EMBED]
    image_item_feats = dpack[:, 0:EMBED]
    text_item_feats = dpack[:, EMBED:2 * EMBED]

    return (u_g, i_g, image_item_feats, text_item_feats, image_user_feats,
            text_user_feats, u_g, i_g, image_user_id, text_user_id)
